# baseline (speedup 1.0000x reference)
import functools

import jax
import jax.numpy as jnp
from jax import lax
from jax.experimental import pallas as pl
from jax.experimental.pallas import tpu as pltpu
from jax.experimental.pallas import tpu_sc as plsc

EMBED = 128
L = 16
NC = 1
NS = 16
NW = NC * NS
CH = 128
NBUF = 2


def kernel(user_input, movie_input, u_table, m_table, fc_w, fc_b):
    B = user_input.shape[0]
    rpw = B // NW
    nch = rpw // CH

    uidx2 = user_input.astype(jnp.int32).reshape(B // CH, CH)
    midx2 = movie_input.astype(jnp.int32).reshape(B // CH, CH)
    fcvec = jnp.zeros((L,), jnp.float32).at[0].set(fc_w[0, 0]).at[1].set(fc_b[0])

    mesh = plsc.VectorSubcoreMesh(
        core_axis_name="c", subcore_axis_name="s", num_cores=NC)

    @functools.partial(
        pl.kernel,
        mesh=mesh,
        out_type=jax.ShapeDtypeStruct((B,), jnp.float32),
        scratch_types=[
            pltpu.VMEM((nch, CH), jnp.int32),
            pltpu.VMEM((nch, CH), jnp.int32),
            pltpu.VMEM((NBUF, CH, EMBED), jnp.float32),
            pltpu.VMEM((NBUF, CH, EMBED), jnp.float32),
            pltpu.VMEM((rpw,), jnp.float32),
            pltpu.VMEM((L,), jnp.float32),
            pltpu.SemaphoreType.DMA,
        ],
    )
    def k(uix_hbm, mix_hbm, ut_hbm, mt_hbm, fc_hbm, out_hbm,
          uidx_v, midx_v, u_v, m_v, o_v, fc_v, sem):
        wid = lax.axis_index("s") * NC + lax.axis_index("c")
        crow = wid * nch
        pltpu.sync_copy(uix_hbm.at[pl.ds(crow, nch)], uidx_v)
        pltpu.sync_copy(mix_hbm.at[pl.ds(crow, nch)], midx_v)

        def fire(s):
            buf = s % NBUF
            cu = pltpu.async_copy(ut_hbm.at[uidx_v.at[s]], u_v.at[buf], sem)
            cm = pltpu.async_copy(mt_hbm.at[midx_v.at[s]], m_v.at[buf], sem)
            return cu, cm

        depth = min(NBUF, nch)
        pending = {s: fire(s) for s in range(depth)}
        pltpu.sync_copy(fc_hbm, fc_v)
        iota = lax.iota(jnp.int32, L)
        fcv = fc_v[...]
        fw = fcv[0]
        fb = fcv[1]
        for s in range(nch):
            pending[s][0].wait()
            pending[s][1].wait()
            buf = s % NBUF
            ub = u_v.at[buf]
            mb = m_v.at[buf]

            def group_body(g, carry):
                res = jnp.zeros((L,), jnp.float32)
                for i in range(L):
                    r = g * L + i
                    acc = ub[r, pl.ds(0, L)] * mb[r, pl.ds(0, L)]
                    for j in range(1, EMBED // L):
                        acc = acc + ub[r, pl.ds(L * j, L)] * mb[r, pl.ds(L * j, L)]
                    for kk in (1, 2, 4, 8):
                        perm = jnp.bitwise_xor(iota, kk)
                        acc = acc + acc.at[perm].get(mode="promise_in_bounds")
                    res = jnp.where(iota == i, acc, res)
                y = res * fw + fb
                o = 1.0 / (1.0 + jnp.exp(-y))
                o_v[pl.ds(s * CH + g * L, L)] = o
                return carry

            lax.fori_loop(0, CH // L, group_body, 0)
            if s + depth < nch:
                pending[s + depth] = fire(s + depth)
        pltpu.sync_copy(o_v, out_hbm.at[pl.ds(wid * rpw, rpw)])

    out = k(uidx2, midx2, u_table, m_table, fcvec)
    return out.reshape(B, 1)


# speedup vs baseline: 1.5482x; 1.5482x over previous
import functools

import jax
import jax.numpy as jnp
from jax import lax
from jax.experimental import pallas as pl
from jax.experimental.pallas import tpu as pltpu
from jax.experimental.pallas import tpu_sc as plsc

EMBED = 128
L = 16
NC = 2
NS = 16
NW = NC * NS
CH = 128
NBUF = 3


def kernel(user_input, movie_input, u_table, m_table, fc_w, fc_b):
    B = user_input.shape[0]
    rpw = B // NW
    nch = rpw // CH

    uidx2 = user_input.astype(jnp.int32).reshape(B // CH, CH)
    midx2 = movie_input.astype(jnp.int32).reshape(B // CH, CH)
    fcvec = jnp.zeros((L,), jnp.float32).at[0].set(fc_w[0, 0]).at[1].set(fc_b[0])

    mesh = plsc.VectorSubcoreMesh(
        core_axis_name="c", subcore_axis_name="s", num_cores=NC)

    @functools.partial(
        pl.kernel,
        mesh=mesh,
        out_type=jax.ShapeDtypeStruct((B,), jnp.float32),
        scratch_types=[
            pltpu.VMEM((nch, CH), jnp.int32),
            pltpu.VMEM((nch, CH), jnp.int32),
            pltpu.VMEM((NBUF, CH, EMBED), jnp.float32),
            pltpu.VMEM((NBUF, CH, EMBED), jnp.float32),
            pltpu.VMEM((rpw,), jnp.float32),
            pltpu.VMEM((L,), jnp.float32),
            pltpu.SemaphoreType.DMA,
        ],
    )
    def k(uix_hbm, mix_hbm, ut_hbm, mt_hbm, fc_hbm, out_hbm,
          uidx_v, midx_v, u_v, m_v, o_v, fc_v, sem):
        wid = lax.axis_index("s") * NC + lax.axis_index("c")
        crow = wid * nch
        pltpu.sync_copy(uix_hbm.at[pl.ds(crow, nch)], uidx_v)
        pltpu.sync_copy(mix_hbm.at[pl.ds(crow, nch)], midx_v)

        def fire(s):
            buf = s % NBUF
            cu = pltpu.async_copy(ut_hbm.at[uidx_v.at[s]], u_v.at[buf], sem)
            cm = pltpu.async_copy(mt_hbm.at[midx_v.at[s]], m_v.at[buf], sem)
            return cu, cm

        depth = min(NBUF, nch)
        pending = {s: fire(s) for s in range(depth)}
        pltpu.sync_copy(fc_hbm, fc_v)
        iota = lax.iota(jnp.int32, L)
        fcv = fc_v[...]
        fw = fcv[0]
        fb = fcv[1]
        for s in range(nch):
            pending[s][0].wait()
            pending[s][1].wait()
            buf = s % NBUF
            ub = u_v.at[buf]
            mb = m_v.at[buf]

            def group_body(g, carry):
                res = jnp.zeros((L,), jnp.float32)
                for i in range(L):
                    r = g * L + i
                    acc = ub[r, pl.ds(0, L)] * mb[r, pl.ds(0, L)]
                    for j in range(1, EMBED // L):
                        acc = acc + ub[r, pl.ds(L * j, L)] * mb[r, pl.ds(L * j, L)]
                    for kk in (1, 2, 4, 8):
                        perm = jnp.bitwise_xor(iota, kk)
                        acc = acc + acc.at[perm].get(mode="promise_in_bounds")
                    res = jnp.where(iota == i, acc, res)
                y = res * fw + fb
                o = 1.0 / (1.0 + jnp.exp(-y))
                o_v[pl.ds(s * CH + g * L, L)] = o
                return carry

            lax.fori_loop(0, CH // L, group_body, 0)
            if s + depth < nch:
                pending[s + depth] = fire(s + depth)
        pltpu.sync_copy(o_v, out_hbm.at[pl.ds(wid * rpw, rpw)])

    out = k(uidx2, midx2, u_table, m_table, fcvec)
    return out.reshape(B, 1)


# speedup vs baseline: 1.5671x; 1.0122x over previous
import functools

import jax
import jax.numpy as jnp
from jax import lax
from jax.experimental import pallas as pl
from jax.experimental.pallas import tpu as pltpu
from jax.experimental.pallas import tpu_sc as plsc

EMBED = 128
L = 16
NC = 2
NS = 16
NW = NC * NS
CH = 128
NBUF = 3


def kernel(user_input, movie_input, u_table, m_table, fc_w, fc_b):
    B = user_input.shape[0]
    rpw = B // NW
    nch = rpw // CH

    uidx2 = user_input.astype(jnp.int32).reshape(B // CH, CH)
    midx2 = movie_input.astype(jnp.int32).reshape(B // CH, CH)
    fcvec = jnp.zeros((L,), jnp.float32).at[0].set(fc_w[0, 0]).at[1].set(fc_b[0])

    mesh = plsc.VectorSubcoreMesh(
        core_axis_name="c", subcore_axis_name="s", num_cores=NC)

    @functools.partial(
        pl.kernel,
        mesh=mesh,
        out_type=jax.ShapeDtypeStruct((B,), jnp.float32),
        scratch_types=[
            pltpu.VMEM((nch, CH), jnp.int32),
            pltpu.VMEM((nch, CH), jnp.int32),
            pltpu.VMEM((NBUF, CH, EMBED), jnp.float32),
            pltpu.VMEM((NBUF, CH, EMBED), jnp.float32),
            pltpu.VMEM((rpw,), jnp.float32),
            pltpu.VMEM((L,), jnp.float32),
            pltpu.SemaphoreType.DMA,
        ],
    )
    def k(uix_hbm, mix_hbm, ut_hbm, mt_hbm, fc_hbm, out_hbm,
          uidx_v, midx_v, u_v, m_v, o_v, fc_v, sem):
        wid = lax.axis_index("s") * NC + lax.axis_index("c")
        crow = wid * nch
        icu = pltpu.async_copy(uix_hbm.at[pl.ds(crow, nch)], uidx_v, sem)
        icm = pltpu.async_copy(mix_hbm.at[pl.ds(crow, nch)], midx_v, sem)
        icu.wait()
        icm.wait()

        def fire(s):
            buf = s % NBUF
            cu = pltpu.async_copy(ut_hbm.at[uidx_v.at[s]], u_v.at[buf], sem)
            cm = pltpu.async_copy(mt_hbm.at[midx_v.at[s]], m_v.at[buf], sem)
            return cu, cm

        depth = min(NBUF, nch)
        pending = {s: fire(s) for s in range(depth)}
        pltpu.sync_copy(fc_hbm, fc_v)
        iota = lax.iota(jnp.int32, L)
        fcv = fc_v[...]
        fw = fcv[0]
        fb = fcv[1]
        for s in range(nch):
            pending[s][0].wait()
            pending[s][1].wait()
            buf = s % NBUF
            ub = u_v.at[buf]
            mb = m_v.at[buf]

            def group_body(g, carry):
                res = jnp.zeros((L,), jnp.float32)
                for i in range(L):
                    r = g * L + i
                    acc = ub[r, pl.ds(0, L)] * mb[r, pl.ds(0, L)]
                    for j in range(1, EMBED // L):
                        acc = acc + ub[r, pl.ds(L * j, L)] * mb[r, pl.ds(L * j, L)]
                    for kk in (1, 2, 4, 8):
                        perm = jnp.bitwise_xor(iota, kk)
                        acc = acc + acc.at[perm].get(mode="promise_in_bounds")
                    res = jnp.where(iota == i, acc, res)
                y = res * fw + fb
                o = 1.0 / (1.0 + jnp.exp(-y))
                o_v[pl.ds(s * CH + g * L, L)] = o
                return carry

            lax.fori_loop(0, CH // L, group_body, 0)
            if s + depth < nch:
                pending[s + depth] = fire(s + depth)
        pltpu.sync_copy(o_v, out_hbm.at[pl.ds(wid * rpw, rpw)])

    out = k(uidx2, midx2, u_table, m_table, fcvec)
    return out.reshape(B, 1)


# speedup vs baseline: 1.5839x; 1.0108x over previous
import functools

import jax
import jax.numpy as jnp
from jax import lax
from jax.experimental import pallas as pl
from jax.experimental.pallas import tpu as pltpu
from jax.experimental.pallas import tpu_sc as plsc

EMBED = 128
L = 16
NC = 2
NS = 16
NW = NC * NS
CH = 128
NBUF = 3


def kernel(user_input, movie_input, u_table, m_table, fc_w, fc_b):
    B = user_input.shape[0]
    rpw = B // NW
    nch = rpw // CH

    uidx2 = user_input.astype(jnp.int32).reshape(B // CH, CH)
    midx2 = movie_input.astype(jnp.int32).reshape(B // CH, CH)
    fcvec = jnp.zeros((L,), jnp.float32).at[0].set(fc_w[0, 0]).at[1].set(fc_b[0])

    mesh = plsc.VectorSubcoreMesh(
        core_axis_name="c", subcore_axis_name="s", num_cores=NC)

    @functools.partial(
        pl.kernel,
        mesh=mesh,
        out_type=jax.ShapeDtypeStruct((B,), jnp.float32),
        scratch_types=[
            pltpu.VMEM((nch, CH), jnp.int32),
            pltpu.VMEM((nch, CH), jnp.int32),
            pltpu.VMEM((NBUF, CH, EMBED), jnp.float32),
            pltpu.VMEM((NBUF, CH, EMBED), jnp.float32),
            pltpu.VMEM((rpw,), jnp.float32),
            pltpu.VMEM((L,), jnp.float32),
            pltpu.SemaphoreType.DMA,
        ],
    )
    def k(uix_hbm, mix_hbm, ut_hbm, mt_hbm, fc_hbm, out_hbm,
          uidx_v, midx_v, u_v, m_v, o_v, fc_v, sem):
        wid = lax.axis_index("s") * NC + lax.axis_index("c")
        crow = wid * nch
        icu = pltpu.async_copy(uix_hbm.at[pl.ds(crow, nch)], uidx_v, sem)
        icm = pltpu.async_copy(mix_hbm.at[pl.ds(crow, nch)], midx_v, sem)
        icu.wait()
        icm.wait()

        def fire(s):
            buf = s % NBUF
            cu = pltpu.async_copy(ut_hbm.at[uidx_v.at[s]], u_v.at[buf], sem)
            cm = pltpu.async_copy(mt_hbm.at[midx_v.at[s]], m_v.at[buf], sem)
            return cu, cm

        depth = min(NBUF, nch)
        pending = {s: fire(s) for s in range(depth)}
        fcc = pltpu.async_copy(fc_hbm, fc_v, sem)
        iota = lax.iota(jnp.int32, L)
        fcc.wait()
        fcv = fc_v[...]
        fw = fcv[0]
        fb = fcv[1]
        for s in range(nch):
            pending[s][0].wait()
            pending[s][1].wait()
            buf = s % NBUF
            ub = u_v.at[buf]
            mb = m_v.at[buf]

            def group_body(g, carry):
                res = jnp.zeros((L,), jnp.float32)
                for i in range(L):
                    r = g * L + i
                    acc = ub[r, pl.ds(0, L)] * mb[r, pl.ds(0, L)]
                    for j in range(1, EMBED // L):
                        acc = acc + ub[r, pl.ds(L * j, L)] * mb[r, pl.ds(L * j, L)]
                    for kk in (1, 2, 4, 8):
                        perm = jnp.bitwise_xor(iota, kk)
                        acc = acc + acc.at[perm].get(mode="promise_in_bounds")
                    res = jnp.where(iota == i, acc, res)
                y = res * fw + fb
                o = 1.0 / (1.0 + jnp.exp(-y))
                o_v[pl.ds(s * CH + g * L, L)] = o
                return carry

            lax.fori_loop(0, CH // L, group_body, 0)
            if s + depth < nch:
                pending[s + depth] = fire(s + depth)
        pltpu.sync_copy(o_v, out_hbm.at[pl.ds(wid * rpw, rpw)])

    out = k(uidx2, midx2, u_table, m_table, fcvec)
    return out.reshape(B, 1)
